# baseline (speedup 1.0000x reference)
import functools

import jax
import jax.numpy as jnp
from jax import lax
from jax.experimental import pallas as pl
from jax.experimental.pallas import tpu as pltpu
from jax.experimental.pallas import tpu_sc as plsc

CODEBOOK = 1024
DIM = 256
TOKENS = 8192
BLK = 1024

_NC = 2
_NS = 16
_NW = _NC * _NS
_BPW = TOKENS // _NW
_CHUNK = 128
_NCHUNK = _BPW // _CHUNK


SUB = 128
NSUB = BLK // SUB


def _vq_block(x_ref, cb_ref, b2_ref, idx_ref, enc_ref):
    cb = cb_ref[...]
    b2 = b2_ref[...]
    lanes = lax.broadcasted_iota(jnp.int32, (SUB, CODEBOOK), 1)

    def dist(t):
        xb = x_ref[pl.ds(t * SUB, SUB), :]
        ab = lax.dot_general(
            xb, cb, (((1,), (1,)), ((), ())),
            precision=lax.Precision.HIGHEST,
            preferred_element_type=jnp.float32,
        )
        a2 = jnp.sum(xb * xb, axis=1, keepdims=True)
        return a2 - 2.0 * ab + b2

    d_prev = dist(0)
    for t in range(1, NSUB + 1):
        d_next = dist(t) if t < NSUB else None
        m = jnp.min(d_prev, axis=1, keepdims=True)
        idx = jnp.min(jnp.where(d_prev == m, lanes, CODEBOOK), axis=1)
        base = (t - 1) * SUB
        enc_ref[pl.ds(base, SUB), :] = (lanes == idx[:, None]).astype(jnp.float32)
        idx_ref[0, 0, pl.ds(base, SUB)] = idx
        d_prev = d_next


_distances_argmin = pl.pallas_call(
    _vq_block,
    grid=(TOKENS // BLK,),
    in_specs=[
        pl.BlockSpec((BLK, DIM), lambda i: (i, 0)),
        pl.BlockSpec((CODEBOOK, DIM), lambda i: (0, 0)),
        pl.BlockSpec((1, CODEBOOK), lambda i: (0, 0)),
    ],
    out_specs=[
        pl.BlockSpec((1, 1, BLK), lambda i: (i, 0, 0)),
        pl.BlockSpec((BLK, CODEBOOK), lambda i: (i, 0)),
    ],
    out_shape=[
        jax.ShapeDtypeStruct((TOKENS // BLK, 1, BLK), jnp.int32),
        jax.ShapeDtypeStruct((TOKENS, CODEBOOK), jnp.float32),
    ],
)


@functools.cache
def _sc_gather_call():
    @functools.partial(
        pl.kernel,
        out_type=jax.ShapeDtypeStruct((_NW * _NCHUNK, _CHUNK, DIM), jnp.float32),
        mesh=plsc.VectorSubcoreMesh(core_axis_name="c", subcore_axis_name="s"),
        scratch_types=[
            pltpu.VMEM((_NCHUNK, _CHUNK), jnp.int32),
            pltpu.VMEM((_NCHUNK, _CHUNK, DIM), jnp.float32),
            pltpu.SemaphoreType.DMA,
        ],
    )
    def _sc_gather(cb_hbm, idx_hbm, out_hbm, idx_v, rows_v, sem):
        wid = lax.axis_index("s") * _NC + lax.axis_index("c")
        base = wid * _NCHUNK
        pltpu.sync_copy(idx_hbm.at[pl.ds(base, _NCHUNK)], idx_v)
        copies = [
            pltpu.async_copy(cb_hbm.at[idx_v.at[j]], rows_v.at[j], sem)
            for j in range(_NCHUNK)
        ]
        for c in copies:
            c.wait()
        pltpu.sync_copy(rows_v, out_hbm.at[pl.ds(base, _NCHUNK)])

    return _sc_gather


def kernel(x, codebook):
    cb = jnp.asarray(codebook, jnp.float32)
    flat = x.reshape(TOKENS, DIM)
    b2 = jnp.sum(cb.T ** 2, axis=0, keepdims=True)
    idx3, enc = _distances_argmin(flat, cb, b2)
    idx_flat = idx3.reshape(TOKENS)
    quant = _sc_gather_call()(cb, idx_flat.reshape(_NW * _NCHUNK, _CHUNK))
    return (
        quant.reshape(x.shape),
        idx_flat.reshape(x.shape[:-1]),
        enc.reshape(x.shape[:-1] + (CODEBOOK,)),
    )


# speedup vs baseline: 1.6444x; 1.6444x over previous
import functools

import jax
import jax.numpy as jnp
from jax import lax
from jax.experimental import pallas as pl
from jax.experimental.pallas import tpu as pltpu
from jax.experimental.pallas import tpu_sc as plsc

CODEBOOK = 1024
DIM = 256
TOKENS = 8192
BLK = 2048

_NC = 2
_NS = 16
_NW = _NC * _NS
_BPW = TOKENS // _NW
_CHUNK = 128
_NCHUNK = _BPW // _CHUNK


SUB = 256
NSUB = BLK // SUB


def _vq_block(x_ref, cb_ref, b2_ref, idx_ref, enc_ref):
    cb = cb_ref[...]
    b2 = b2_ref[...]
    lanes = lax.broadcasted_iota(jnp.int32, (SUB, CODEBOOK), 1)

    def dist(t):
        xb = x_ref[pl.ds(t * SUB, SUB), :]
        ab = lax.dot_general(
            xb, cb, (((1,), (1,)), ((), ())),
            precision=lax.Precision.HIGHEST,
            preferred_element_type=jnp.float32,
        )
        a2 = jnp.sum(xb * xb, axis=1, keepdims=True)
        return a2 - 2.0 * ab + b2

    d_prev = dist(0)
    for t in range(1, NSUB + 1):
        d_next = dist(t) if t < NSUB else None
        m = jnp.min(d_prev, axis=1, keepdims=True)
        idx = jnp.min(jnp.where(d_prev == m, lanes, CODEBOOK), axis=1)
        base = (t - 1) * SUB
        enc_ref[pl.ds(base, SUB), :] = (lanes == idx[:, None]).astype(jnp.float32)
        idx_ref[0, 0, pl.ds(base, SUB)] = idx
        d_prev = d_next


_distances_argmin = pl.pallas_call(
    _vq_block,
    grid=(TOKENS // BLK,),
    in_specs=[
        pl.BlockSpec((BLK, DIM), lambda i: (i, 0)),
        pl.BlockSpec((CODEBOOK, DIM), lambda i: (0, 0)),
        pl.BlockSpec((1, CODEBOOK), lambda i: (0, 0)),
    ],
    out_specs=[
        pl.BlockSpec((1, 1, BLK), lambda i: (i, 0, 0)),
        pl.BlockSpec((BLK, CODEBOOK), lambda i: (i, 0)),
    ],
    out_shape=[
        jax.ShapeDtypeStruct((TOKENS // BLK, 1, BLK), jnp.int32),
        jax.ShapeDtypeStruct((TOKENS, CODEBOOK), jnp.float32),
    ],
)


@functools.cache
def _sc_gather_call():
    @functools.partial(
        pl.kernel,
        out_type=jax.ShapeDtypeStruct((_NW * _NCHUNK, _CHUNK, DIM), jnp.float32),
        mesh=plsc.VectorSubcoreMesh(core_axis_name="c", subcore_axis_name="s"),
        scratch_types=[
            pltpu.VMEM((_NCHUNK, _CHUNK), jnp.int32),
            pltpu.VMEM((_NCHUNK, _CHUNK, DIM), jnp.float32),
            pltpu.SemaphoreType.DMA,
        ],
    )
    def _sc_gather(cb_hbm, idx_hbm, out_hbm, idx_v, rows_v, sem):
        wid = lax.axis_index("s") * _NC + lax.axis_index("c")
        base = wid * _NCHUNK
        pltpu.sync_copy(idx_hbm.at[pl.ds(base, _NCHUNK)], idx_v)
        copies = [
            pltpu.async_copy(cb_hbm.at[idx_v.at[j]], rows_v.at[j], sem)
            for j in range(_NCHUNK)
        ]
        for c in copies:
            c.wait()
        pltpu.sync_copy(rows_v, out_hbm.at[pl.ds(base, _NCHUNK)])

    return _sc_gather


def kernel(x, codebook):
    cb = jnp.asarray(codebook, jnp.float32)
    flat = x.reshape(TOKENS, DIM)
    b2 = jnp.sum(cb.T ** 2, axis=0, keepdims=True)
    idx3, enc = _distances_argmin(flat, cb, b2)
    idx_flat = idx3.reshape(TOKENS)
    quant = _sc_gather_call()(cb, idx_flat.reshape(_NW * _NCHUNK, _CHUNK))
    return (
        quant.reshape(x.shape),
        idx_flat.reshape(x.shape[:-1]),
        enc.reshape(x.shape[:-1] + (CODEBOOK,)),
    )
